# Initial kernel scaffold; baseline (speedup 1.0000x reference)
import jax
import jax.numpy as jnp
from jax.experimental import pallas as pl


def kernel(x, edge_index, edge_type, W_conv1, g1, b1, W_enc, g2, b2, W_down, g3, b3, W_pre, b_pre, codebook, W_post, b_post, W_dec1, g4, b4, W_dec2, g5, b5, W_reg1, b_reg1, W_reg2, b_reg2):
    raise NotImplementedError("write your pallas kernel here")


# speedup vs baseline: 10.4150x; 10.4150x over previous
import functools

import jax
import jax.numpy as jnp
from jax import lax
from jax.experimental import pallas as pl
from jax.experimental.pallas import tpu as pltpu
from jax.experimental.pallas import tpu_sc as plsc

_N = 50000
_E = 800000
_NET = 7
_NSUB = 16
_STREAM = 128
_SPS = 392
_EPAD = 2 * _NSUB * _SPS * _STREAM // 2
_EPS = _N + _STREAM
_RS_A = 3128
_RS_LAST = _N - 15 * _RS_A
_F32 = jnp.float32
_R = 2000
_GRID = _N // _R


def _gidx_body(src_ref, et_ref, out_ref):
    out_ref[...] = src_ref[...] * _NET + et_ref[...]


def _compute_gidx(srcp, etp):
    v = srcp.reshape(_EPAD // 128, 128)
    e = etp.reshape(_EPAD // 128, 128)
    out = pl.pallas_call(
        _gidx_body,
        out_shape=jax.ShapeDtypeStruct(v.shape, jnp.int32),
    )(v, e)
    return out.reshape(_EPAD)


def _pairsum(row, gsz):
    c = row.shape[1]
    ii = lax.broadcasted_iota(jnp.int32, (c, c), 0) // gsz
    jj = lax.broadcasted_iota(jnp.int32, (c, c), 1) // gsz
    return _xdot(row, (ii == jj).astype(_F32))


def _stats_body(y_ref, sum_ref):
    @pl.when(pl.program_id(0) == 0)
    def _():
        sum_ref[...] = jnp.zeros_like(sum_ref)

    sum_ref[...] += jnp.sum(y_ref[...], axis=0, keepdims=True)


def _stats(y, gsz):
    n, c = y.shape
    ssum = pl.pallas_call(
        _stats_body,
        grid=(n // _R,),
        in_specs=[pl.BlockSpec((_R, c), lambda i: (i, 0))],
        out_specs=pl.BlockSpec((1, c), lambda i: (0, 0)),
        out_shape=jax.ShapeDtypeStruct((1, c), _F32),
    )(y)

    def _sq_body(y_ref, s_ref, sq_ref):
        @pl.when(pl.program_id(0) == 0)
        def _():
            sq_ref[...] = jnp.zeros_like(sq_ref)

        mu = _pairsum(s_ref[...], gsz) / float(_N * gsz)
        dlt = y_ref[...] - mu
        sq_ref[...] += jnp.sum(dlt * dlt, axis=0, keepdims=True)

    ssq = pl.pallas_call(
        _sq_body,
        grid=(n // _R,),
        in_specs=[
            pl.BlockSpec((_R, c), lambda i: (i, 0)),
            pl.BlockSpec((1, c), lambda i: (0, 0)),
        ],
        out_specs=pl.BlockSpec((1, c), lambda i: (0, 0)),
        out_shape=jax.ShapeDtypeStruct((1, c), _F32),
    )(y, ssum)
    return ssum, ssq


def _bdot(a, b):
    return jnp.dot(a.astype(jnp.bfloat16), b.astype(jnp.bfloat16),
                   preferred_element_type=_F32)


def _xdot(a, b, dims=None):
    if dims is None:
        return jnp.dot(a, b, precision=lax.Precision.HIGHEST,
                       preferred_element_type=_F32)
    return lax.dot_general(a, b, (dims, ((), ())),
                           precision=lax.Precision.HIGHEST,
                           preferred_element_type=_F32)


def _gn_relu(blk, ssum, ssq, gamma, beta):
    c = blk.shape[1]
    gsz = c // 32
    cnt = float(_N * gsz)
    mu = _pairsum(ssum, gsz) / cnt
    var = _pairsum(ssq, gsz) / cnt
    return jnp.maximum(
        (blk - mu) / jnp.sqrt(var + 1e-5) * gamma + beta, 0.0)


def _mm_body(h_ref, w0_ref, w1_ref, x0_ref, x1_ref):
    h = h_ref[...]
    x0_ref[...] = _bdot(h, w0_ref[...])
    x1_ref[...] = _bdot(h, w1_ref[...])


def _conv_in_plain(h, w0, w1):
    cin, m = w0.shape
    return pl.pallas_call(
        _mm_body,
        grid=(_GRID,),
        in_specs=[
            pl.BlockSpec((_R, cin), lambda i: (i, 0)),
            pl.BlockSpec((cin, m), lambda i: (0, 0)),
            pl.BlockSpec((cin, m), lambda i: (0, 0)),
        ],
        out_specs=[pl.BlockSpec((_R, m), lambda i: (i, 0))] * 2,
        out_shape=[jax.ShapeDtypeStruct((_N, m), _F32)] * 2,
    )(h, w0, w1)


def _gn_mm_body(ya_ref, yb_ref, sa_ref, qa_ref, sb_ref, qb_ref, g_ref, b_ref,
                w0_ref, w1_ref, x0_ref, x1_ref, h_ref):
    y = jnp.concatenate([ya_ref[...], yb_ref[...]], axis=1)
    ssum = jnp.concatenate([sa_ref[...], sb_ref[...]], axis=1)
    ssq = jnp.concatenate([qa_ref[...], qb_ref[...]], axis=1)
    h = _gn_relu(y, ssum, ssq, g_ref[...], b_ref[...])
    h_ref[...] = h
    x0_ref[...] = _bdot(h, w0_ref[...])
    x1_ref[...] = _bdot(h, w1_ref[...])


def _conv_in_gn(ya, yb, sa, qa, sb, qb, g, b, w0, w1):
    cin, m = w0.shape
    ch = cin // 2
    return pl.pallas_call(
        _gn_mm_body,
        grid=(_GRID,),
        in_specs=[
            pl.BlockSpec((_R, ch), lambda i: (i, 0)),
            pl.BlockSpec((_R, ch), lambda i: (i, 0)),
            pl.BlockSpec((1, ch), lambda i: (0, 0)),
            pl.BlockSpec((1, ch), lambda i: (0, 0)),
            pl.BlockSpec((1, ch), lambda i: (0, 0)),
            pl.BlockSpec((1, ch), lambda i: (0, 0)),
            pl.BlockSpec((1, cin), lambda i: (0, 0)),
            pl.BlockSpec((1, cin), lambda i: (0, 0)),
            pl.BlockSpec((cin, m), lambda i: (0, 0)),
            pl.BlockSpec((cin, m), lambda i: (0, 0)),
        ],
        out_specs=[
            pl.BlockSpec((_R, m), lambda i: (i, 0)),
            pl.BlockSpec((_R, m), lambda i: (i, 0)),
            pl.BlockSpec((_R, cin), lambda i: (i, 0)),
        ],
        out_shape=[
            jax.ShapeDtypeStruct((_N, m), _F32),
            jax.ShapeDtypeStruct((_N, m), _F32),
            jax.ShapeDtypeStruct((_N, cin), _F32),
        ],
    )(ya, yb, sa, qa, sb, qb, g, b, w0, w1)


def _res_mm_body(hp_ref, ya_ref, yb_ref, sa_ref, qa_ref, sb_ref, qb_ref,
                 g_ref, b_ref, w0_ref, w1_ref, x0_ref, x1_ref):
    y = jnp.concatenate([ya_ref[...], yb_ref[...]], axis=1)
    ssum = jnp.concatenate([sa_ref[...], sb_ref[...]], axis=1)
    ssq = jnp.concatenate([qa_ref[...], qb_ref[...]], axis=1)
    h = hp_ref[...] + _gn_relu(y, ssum, ssq, g_ref[...], b_ref[...])
    x0_ref[...] = _bdot(h, w0_ref[...])
    x1_ref[...] = _bdot(h, w1_ref[...])


def _conv_in_res(hp, ya, yb, sa, qa, sb, qb, g, b, w0, w1):
    cin, m = w0.shape
    ch = cin // 2
    return pl.pallas_call(
        _res_mm_body,
        grid=(_GRID,),
        in_specs=[
            pl.BlockSpec((_R, cin), lambda i: (i, 0)),
            pl.BlockSpec((_R, ch), lambda i: (i, 0)),
            pl.BlockSpec((_R, ch), lambda i: (i, 0)),
            pl.BlockSpec((1, ch), lambda i: (0, 0)),
            pl.BlockSpec((1, ch), lambda i: (0, 0)),
            pl.BlockSpec((1, ch), lambda i: (0, 0)),
            pl.BlockSpec((1, ch), lambda i: (0, 0)),
            pl.BlockSpec((1, cin), lambda i: (0, 0)),
            pl.BlockSpec((1, cin), lambda i: (0, 0)),
            pl.BlockSpec((cin, m), lambda i: (0, 0)),
            pl.BlockSpec((cin, m), lambda i: (0, 0)),
        ],
        out_specs=[pl.BlockSpec((_R, m), lambda i: (i, 0))] * 2,
        out_shape=[jax.ShapeDtypeStruct((_N, m), _F32)] * 2,
    )(hp, ya, yb, sa, qa, sb, qb, g, b, w0, w1)


def _vq_body(ya_ref, yb_ref, sa_ref, qa_ref, sb_ref, qb_ref, g_ref, b_ref,
             wpre_ref, bpre_ref, cb_ref, c2_ref, wpost_ref, bp_ref, out_ref):
    y = jnp.concatenate([ya_ref[...], yb_ref[...]], axis=1)
    ssum = jnp.concatenate([sa_ref[...], sb_ref[...]], axis=1)
    ssq = jnp.concatenate([qa_ref[...], qb_ref[...]], axis=1)
    h3 = _gn_relu(y, ssum, ssq, g_ref[...], b_ref[...])
    cb = cb_ref[...]
    z = _bdot(h3, wpre_ref[...]) + bpre_ref[...]
    zc = lax.dot_general(
        z.astype(jnp.bfloat16), cb.astype(jnp.bfloat16),
        (((1,), (1,)), ((), ())), preferred_element_type=_F32)
    d = jnp.sum(z * z, axis=1, keepdims=True) - 2.0 * zc + c2_ref[...]
    mn = jnp.min(d, axis=1, keepdims=True)
    iotaj = lax.broadcasted_iota(jnp.int32, d.shape, 1).astype(_F32)
    first = jnp.min(jnp.where(d == mn, iotaj, 1e9), axis=1, keepdims=True)
    onehot = (iotaj == first).astype(_F32)
    zq = _bdot(onehot, cb)
    out_ref[...] = _bdot(zq, wpost_ref[...]) + bp_ref[...]


def _vq(ya, yb, sa, qa, sb, qb, g, b, wpre, bpre, cb, c2, wpost, bpost):
    c1, emb = wpre.shape
    k = cb.shape[0]
    ch = c1 // 2
    return pl.pallas_call(
        _vq_body,
        grid=(_GRID,),
        in_specs=[
            pl.BlockSpec((_R, ch), lambda i: (i, 0)),
            pl.BlockSpec((_R, ch), lambda i: (i, 0)),
            pl.BlockSpec((1, ch), lambda i: (0, 0)),
            pl.BlockSpec((1, ch), lambda i: (0, 0)),
            pl.BlockSpec((1, ch), lambda i: (0, 0)),
            pl.BlockSpec((1, ch), lambda i: (0, 0)),
            pl.BlockSpec((1, c1), lambda i: (0, 0)),
            pl.BlockSpec((1, c1), lambda i: (0, 0)),
            pl.BlockSpec((c1, emb), lambda i: (0, 0)),
            pl.BlockSpec((1, emb), lambda i: (0, 0)),
            pl.BlockSpec((k, emb), lambda i: (0, 0)),
            pl.BlockSpec((1, k), lambda i: (0, 0)),
            pl.BlockSpec((emb, c1), lambda i: (0, 0)),
            pl.BlockSpec((1, c1), lambda i: (0, 0)),
        ],
        out_specs=pl.BlockSpec((_R, c1), lambda i: (i, 0)),
        out_shape=jax.ShapeDtypeStruct((_N, c1), _F32),
    )(ya, yb, sa, qa, sb, qb, g, b, wpre, bpre, cb, c2, wpost, bpost)


def _head_body(ya_ref, yb_ref, sa_ref, qa_ref, sb_ref, qb_ref, g_ref, b_ref,
               w1_ref, b1_ref, w2_ref, b2_ref, out_ref):
    y = jnp.concatenate([ya_ref[...], yb_ref[...]], axis=1)
    ssum = jnp.concatenate([sa_ref[...], sb_ref[...]], axis=1)
    ssq = jnp.concatenate([qa_ref[...], qb_ref[...]], axis=1)
    h = _gn_relu(y, ssum, ssq, g_ref[...], b_ref[...])
    mid = jnp.maximum(_bdot(h, w1_ref[...]) + b1_ref[...], 0.0)
    out_ref[...] = _bdot(mid, w2_ref[...]) + b2_ref[...]


def _head(ya, yb, sa, qa, sb, qb, g, b, w1, b1, w2, b2):
    c0 = w1.shape[0]
    mid = w1.shape[1]
    oc = w2.shape[1]
    ch = c0 // 2
    return pl.pallas_call(
        _head_body,
        grid=(_GRID,),
        in_specs=[
            pl.BlockSpec((_R, ch), lambda i: (i, 0)),
            pl.BlockSpec((_R, ch), lambda i: (i, 0)),
            pl.BlockSpec((1, ch), lambda i: (0, 0)),
            pl.BlockSpec((1, ch), lambda i: (0, 0)),
            pl.BlockSpec((1, ch), lambda i: (0, 0)),
            pl.BlockSpec((1, ch), lambda i: (0, 0)),
            pl.BlockSpec((1, c0), lambda i: (0, 0)),
            pl.BlockSpec((1, c0), lambda i: (0, 0)),
            pl.BlockSpec((c0, mid), lambda i: (0, 0)),
            pl.BlockSpec((1, mid), lambda i: (0, 0)),
            pl.BlockSpec((mid, oc), lambda i: (0, 0)),
            pl.BlockSpec((1, oc), lambda i: (0, 0)),
        ],
        out_specs=pl.BlockSpec((_R, oc), lambda i: (i, 0)),
        out_shape=jax.ShapeDtypeStruct((_N, oc), _F32),
    )(ya, yb, sa, qa, sb, qb, g, b, w1, b1, w2, b2)


@functools.cache
def _make_sc_conv(ch):
    mesh = plsc.VectorSubcoreMesh(core_axis_name="c", subcore_axis_name="s")
    grp = 1024 if ch <= 16 else 512
    nmacro = _SPS * _STREAM // grp
    nstream = grp // _STREAM

    @functools.partial(
        pl.kernel,
        out_type=[jax.ShapeDtypeStruct((_N, ch), _F32)] * 2,
        mesh=mesh,
        scratch_types=(
            [pltpu.VMEM((_STREAM,), jnp.int32)] * (2 * nstream)
            + [
                pltpu.VMEM((grp, ch), _F32),
                pltpu.VMEM_SHARED((_EPS, ch), _F32),
                pltpu.SemaphoreType.DMA,
            ]
        ),
        compiler_params=pltpu.CompilerParams(use_tc_tiling_on_sc=False),
    )
    def conv(xw0, xw1, gidx, dst, zrows, out0, out1, *scr):
        idxbs = scr[:nstream]
        dstbs = scr[nstream:2 * nstream]
        rows, acc, sem = scr[2 * nstream:]
        c = lax.axis_index("c")
        s = lax.axis_index("s")
        @pl.when(s < 15)
        def _():
            pltpu.sync_copy(zrows, acc.at[pl.ds(s * _RS_A, _RS_A)])

        @pl.when(s == 15)
        def _():
            pltpu.sync_copy(zrows.at[pl.ds(0, _RS_LAST)],
                            acc.at[pl.ds(15 * _RS_A, _RS_LAST)])

        plsc.subcore_barrier()

        def run(xw):
            def macro(i, carry):
                base = s * (_SPS * _STREAM) + i * grp
                for j in range(nstream):
                    pltpu.sync_copy(
                        gidx.at[pl.ds(base + j * _STREAM, _STREAM)], idxbs[j])
                    pltpu.sync_copy(
                        dst.at[pl.ds(base + j * _STREAM, _STREAM)], dstbs[j])
                cps = [
                    pltpu.async_copy(
                        xw.at[idxbs[j]],
                        rows.at[pl.ds(j * _STREAM, _STREAM)],
                        sem,
                    )
                    for j in range(nstream)
                ]
                for cp in cps:
                    cp.wait()
                for j in range(nstream):
                    pltpu.sync_copy(
                        rows.at[pl.ds(j * _STREAM, _STREAM)],
                        acc.at[dstbs[j]],
                        add=True,
                    )
                return carry
            lax.fori_loop(0, nmacro, macro, 0)

        @pl.when(c == 0)
        def _():
            run(xw0)

        @pl.when(c == 1)
        def _():
            run(xw1)

        plsc.subcore_barrier()

        def copy_out(out):
            @pl.when(s < 15)
            def _():
                pltpu.sync_copy(acc.at[pl.ds(s * _RS_A, _RS_A)],
                                out.at[pl.ds(s * _RS_A, _RS_A)])

            @pl.when(s == 15)
            def _():
                pltpu.sync_copy(acc.at[pl.ds(15 * _RS_A, _RS_LAST)],
                                out.at[pl.ds(15 * _RS_A, _RS_LAST)])

        @pl.when(c == 0)
        def _():
            copy_out(out0)

        @pl.when(c == 1)
        def _():
            copy_out(out1)

    return conv


def _sc_conv(ch, xw0, xw1, gidx, dstp, zrows):
    return _make_sc_conv(ch)(
        xw0.reshape(_N * _NET, ch), xw1.reshape(_N * _NET, ch),
        gidx, dstp, zrows)


def _halves(w):
    net, cin, cout = w.shape
    ch = cout // 2
    wt = jnp.transpose(w, (1, 0, 2))
    return (wt[:, :, :ch].reshape(cin, net * ch),
            wt[:, :, ch:].reshape(cin, net * ch))


def kernel(x, edge_index, edge_type, W_conv1, g1, b1, W_enc, g2, b2,
           W_down, g3, b3, W_pre, b_pre, codebook, W_post, b_post,
           W_dec1, g4, b4, W_dec2, g5, b5, W_reg1, b_reg1, W_reg2, b_reg2):
    src = edge_index[0].astype(jnp.int32)
    dst = edge_index[1].astype(jnp.int32)
    et = edge_type.astype(jnp.int32)

    pad = _EPAD - _E
    srcp = jnp.concatenate([src, jnp.arange(pad, dtype=jnp.int32) % 101])
    etp = jnp.concatenate([et, jnp.zeros((pad,), jnp.int32)])
    dstp = jnp.concatenate(
        [dst, _N + (jnp.arange(pad, dtype=jnp.int32) % _STREAM)])
    gidx = _compute_gidx(srcp, etp)
    z16 = jnp.zeros((_RS_A, 16), _F32)
    z32 = jnp.zeros((_RS_A, 32), _F32)

    r2 = lambda a: a.reshape(1, -1)

    w0, w1 = _halves(W_conv1)
    xw0, xw1 = _conv_in_plain(x, w0, w1)
    y1a, y1b = _sc_conv(16, xw0, xw1, gidx, dstp, z16)
    s1a, q1a = _stats(y1a, 1)
    s1b, q1b = _stats(y1b, 1)

    w0, w1 = _halves(W_enc)
    xw0, xw1, h1 = _conv_in_gn(y1a, y1b, s1a, q1a, s1b, q1b,
                               r2(g1), r2(b1), w0, w1)
    y2a, y2b = _sc_conv(16, xw0, xw1, gidx, dstp, z16)
    s2a, q2a = _stats(y2a, 1)
    s2b, q2b = _stats(y2b, 1)

    w0, w1 = _halves(W_down)
    xw0, xw1 = _conv_in_res(h1, y2a, y2b, s2a, q2a, s2b, q2b,
                            r2(g2), r2(b2), w0, w1)
    y3a, y3b = _sc_conv(32, xw0, xw1, gidx, dstp, z32)
    s3a, q3a = _stats(y3a, 2)
    s3b, q3b = _stats(y3b, 2)

    c2 = jnp.sum(codebook * codebook, axis=1).reshape(1, -1)
    h4 = _vq(y3a, y3b, s3a, q3a, s3b, q3b, r2(g3), r2(b3),
             W_pre, r2(b_pre), codebook, c2, W_post, r2(b_post))

    w0, w1 = _halves(W_dec1)
    xw0, xw1 = _conv_in_plain(h4, w0, w1)
    y4a, y4b = _sc_conv(32, xw0, xw1, gidx, dstp, z32)
    s4a, q4a = _stats(y4a, 2)
    s4b, q4b = _stats(y4b, 2)

    w0, w1 = _halves(W_dec2)
    xw0, xw1 = _conv_in_res(h4, y4a, y4b, s4a, q4a, s4b, q4b,
                            r2(g4), r2(b4), w0, w1)
    y5a, y5b = _sc_conv(16, xw0, xw1, gidx, dstp, z16)
    s5a, q5a = _stats(y5a, 1)
    s5b, q5b = _stats(y5b, 1)

    return _head(y5a, y5b, s5a, q5a, s5b, q5b, r2(g5), r2(b5),
                 W_reg1, r2(b_reg1), W_reg2, r2(b_reg2))


# speedup vs baseline: 13.2988x; 1.2769x over previous
import functools

import jax
import jax.numpy as jnp
from jax import lax
from jax.experimental import pallas as pl
from jax.experimental.pallas import tpu as pltpu
from jax.experimental.pallas import tpu_sc as plsc

_N = 50000
_E = 800000
_NET = 7
_NSUB = 16
_STREAM = 256
_SPS = 196
_EPAD = 2 * _NSUB * _SPS * _STREAM // 2
_EPS = _N + _STREAM
_RS_A = 3128
_RS_LAST = _N - 15 * _RS_A
_F32 = jnp.float32
_R = 2000
_GRID = _N // _R


def _gidx_body(src_ref, et_ref, out_ref):
    out_ref[...] = src_ref[...] * _NET + et_ref[...]


def _compute_gidx(srcp, etp):
    v = srcp.reshape(_EPAD // 128, 128)
    e = etp.reshape(_EPAD // 128, 128)
    out = pl.pallas_call(
        _gidx_body,
        out_shape=jax.ShapeDtypeStruct(v.shape, jnp.int32),
    )(v, e)
    return out.reshape(_EPAD)


def _pairsum(row, gsz):
    c = row.shape[1]
    ii = lax.broadcasted_iota(jnp.int32, (c, c), 0) // gsz
    jj = lax.broadcasted_iota(jnp.int32, (c, c), 1) // gsz
    return _xdot(row, (ii == jj).astype(_F32))


def _stats_body(y_ref, sum_ref):
    @pl.when(pl.program_id(0) == 0)
    def _():
        sum_ref[...] = jnp.zeros_like(sum_ref)

    sum_ref[...] += jnp.sum(y_ref[...], axis=0, keepdims=True)


def _stats(y, gsz):
    n, c = y.shape
    ssum = pl.pallas_call(
        _stats_body,
        grid=(n // _R,),
        in_specs=[pl.BlockSpec((_R, c), lambda i: (i, 0))],
        out_specs=pl.BlockSpec((1, c), lambda i: (0, 0)),
        out_shape=jax.ShapeDtypeStruct((1, c), _F32),
    )(y)

    def _sq_body(y_ref, s_ref, sq_ref):
        @pl.when(pl.program_id(0) == 0)
        def _():
            sq_ref[...] = jnp.zeros_like(sq_ref)

        mu = _pairsum(s_ref[...], gsz) / float(_N * gsz)
        dlt = y_ref[...] - mu
        sq_ref[...] += jnp.sum(dlt * dlt, axis=0, keepdims=True)

    ssq = pl.pallas_call(
        _sq_body,
        grid=(n // _R,),
        in_specs=[
            pl.BlockSpec((_R, c), lambda i: (i, 0)),
            pl.BlockSpec((1, c), lambda i: (0, 0)),
        ],
        out_specs=pl.BlockSpec((1, c), lambda i: (0, 0)),
        out_shape=jax.ShapeDtypeStruct((1, c), _F32),
    )(y, ssum)
    return ssum, ssq


def _bdot(a, b):
    return jnp.dot(a.astype(jnp.bfloat16), b.astype(jnp.bfloat16),
                   preferred_element_type=_F32)


def _xdot(a, b, dims=None):
    if dims is None:
        return jnp.dot(a, b, precision=lax.Precision.HIGHEST,
                       preferred_element_type=_F32)
    return lax.dot_general(a, b, (dims, ((), ())),
                           precision=lax.Precision.HIGHEST,
                           preferred_element_type=_F32)


def _gn_relu(blk, ssum, ssq, gamma, beta):
    c = blk.shape[1]
    gsz = c // 32
    cnt = float(_N * gsz)
    mu = _pairsum(ssum, gsz) / cnt
    var = _pairsum(ssq, gsz) / cnt
    return jnp.maximum(
        (blk - mu) / jnp.sqrt(var + 1e-5) * gamma + beta, 0.0)


def _mm_body(h_ref, w0_ref, w1_ref, x0_ref, x1_ref):
    h = h_ref[...]
    x0_ref[...] = _bdot(h, w0_ref[...])
    x1_ref[...] = _bdot(h, w1_ref[...])


def _conv_in_plain(h, w0, w1):
    cin, m = w0.shape
    return pl.pallas_call(
        _mm_body,
        grid=(_GRID,),
        in_specs=[
            pl.BlockSpec((_R, cin), lambda i: (i, 0)),
            pl.BlockSpec((cin, m), lambda i: (0, 0)),
            pl.BlockSpec((cin, m), lambda i: (0, 0)),
        ],
        out_specs=[pl.BlockSpec((_R, m), lambda i: (i, 0))] * 2,
        out_shape=[jax.ShapeDtypeStruct((_N, m), _F32)] * 2,
    )(h, w0, w1)


def _gn_mm_body(ya_ref, yb_ref, sa_ref, qa_ref, sb_ref, qb_ref, g_ref, b_ref,
                w0_ref, w1_ref, x0_ref, x1_ref, h_ref):
    y = jnp.concatenate([ya_ref[...], yb_ref[...]], axis=1)
    ssum = jnp.concatenate([sa_ref[...], sb_ref[...]], axis=1)
    ssq = jnp.concatenate([qa_ref[...], qb_ref[...]], axis=1)
    h = _gn_relu(y, ssum, ssq, g_ref[...], b_ref[...])
    h_ref[...] = h
    x0_ref[...] = _bdot(h, w0_ref[...])
    x1_ref[...] = _bdot(h, w1_ref[...])


def _conv_in_gn(ya, yb, sa, qa, sb, qb, g, b, w0, w1):
    cin, m = w0.shape
    ch = cin // 2
    return pl.pallas_call(
        _gn_mm_body,
        grid=(_GRID,),
        in_specs=[
            pl.BlockSpec((_R, ch), lambda i: (i, 0)),
            pl.BlockSpec((_R, ch), lambda i: (i, 0)),
            pl.BlockSpec((1, ch), lambda i: (0, 0)),
            pl.BlockSpec((1, ch), lambda i: (0, 0)),
            pl.BlockSpec((1, ch), lambda i: (0, 0)),
            pl.BlockSpec((1, ch), lambda i: (0, 0)),
            pl.BlockSpec((1, cin), lambda i: (0, 0)),
            pl.BlockSpec((1, cin), lambda i: (0, 0)),
            pl.BlockSpec((cin, m), lambda i: (0, 0)),
            pl.BlockSpec((cin, m), lambda i: (0, 0)),
        ],
        out_specs=[
            pl.BlockSpec((_R, m), lambda i: (i, 0)),
            pl.BlockSpec((_R, m), lambda i: (i, 0)),
            pl.BlockSpec((_R, cin), lambda i: (i, 0)),
        ],
        out_shape=[
            jax.ShapeDtypeStruct((_N, m), _F32),
            jax.ShapeDtypeStruct((_N, m), _F32),
            jax.ShapeDtypeStruct((_N, cin), _F32),
        ],
    )(ya, yb, sa, qa, sb, qb, g, b, w0, w1)


def _res_mm_body(hp_ref, ya_ref, yb_ref, sa_ref, qa_ref, sb_ref, qb_ref,
                 g_ref, b_ref, w0_ref, w1_ref, x0_ref, x1_ref):
    y = jnp.concatenate([ya_ref[...], yb_ref[...]], axis=1)
    ssum = jnp.concatenate([sa_ref[...], sb_ref[...]], axis=1)
    ssq = jnp.concatenate([qa_ref[...], qb_ref[...]], axis=1)
    h = hp_ref[...] + _gn_relu(y, ssum, ssq, g_ref[...], b_ref[...])
    x0_ref[...] = _bdot(h, w0_ref[...])
    x1_ref[...] = _bdot(h, w1_ref[...])


def _conv_in_res(hp, ya, yb, sa, qa, sb, qb, g, b, w0, w1):
    cin, m = w0.shape
    ch = cin // 2
    return pl.pallas_call(
        _res_mm_body,
        grid=(_GRID,),
        in_specs=[
            pl.BlockSpec((_R, cin), lambda i: (i, 0)),
            pl.BlockSpec((_R, ch), lambda i: (i, 0)),
            pl.BlockSpec((_R, ch), lambda i: (i, 0)),
            pl.BlockSpec((1, ch), lambda i: (0, 0)),
            pl.BlockSpec((1, ch), lambda i: (0, 0)),
            pl.BlockSpec((1, ch), lambda i: (0, 0)),
            pl.BlockSpec((1, ch), lambda i: (0, 0)),
            pl.BlockSpec((1, cin), lambda i: (0, 0)),
            pl.BlockSpec((1, cin), lambda i: (0, 0)),
            pl.BlockSpec((cin, m), lambda i: (0, 0)),
            pl.BlockSpec((cin, m), lambda i: (0, 0)),
        ],
        out_specs=[pl.BlockSpec((_R, m), lambda i: (i, 0))] * 2,
        out_shape=[jax.ShapeDtypeStruct((_N, m), _F32)] * 2,
    )(hp, ya, yb, sa, qa, sb, qb, g, b, w0, w1)


def _vq_body(ya_ref, yb_ref, sa_ref, qa_ref, sb_ref, qb_ref, g_ref, b_ref,
             wpre_ref, bpre_ref, cb_ref, c2_ref, wpost_ref, bp_ref, out_ref):
    y = jnp.concatenate([ya_ref[...], yb_ref[...]], axis=1)
    ssum = jnp.concatenate([sa_ref[...], sb_ref[...]], axis=1)
    ssq = jnp.concatenate([qa_ref[...], qb_ref[...]], axis=1)
    h3 = _gn_relu(y, ssum, ssq, g_ref[...], b_ref[...])
    cb = cb_ref[...]
    z = _bdot(h3, wpre_ref[...]) + bpre_ref[...]
    zc = lax.dot_general(
        z.astype(jnp.bfloat16), cb.astype(jnp.bfloat16),
        (((1,), (1,)), ((), ())), preferred_element_type=_F32)
    d = jnp.sum(z * z, axis=1, keepdims=True) - 2.0 * zc + c2_ref[...]
    mn = jnp.min(d, axis=1, keepdims=True)
    iotaj = lax.broadcasted_iota(jnp.int32, d.shape, 1).astype(_F32)
    first = jnp.min(jnp.where(d == mn, iotaj, 1e9), axis=1, keepdims=True)
    onehot = (iotaj == first).astype(_F32)
    zq = _bdot(onehot, cb)
    out_ref[...] = _bdot(zq, wpost_ref[...]) + bp_ref[...]


def _vq(ya, yb, sa, qa, sb, qb, g, b, wpre, bpre, cb, c2, wpost, bpost):
    c1, emb = wpre.shape
    k = cb.shape[0]
    ch = c1 // 2
    return pl.pallas_call(
        _vq_body,
        grid=(_GRID,),
        in_specs=[
            pl.BlockSpec((_R, ch), lambda i: (i, 0)),
            pl.BlockSpec((_R, ch), lambda i: (i, 0)),
            pl.BlockSpec((1, ch), lambda i: (0, 0)),
            pl.BlockSpec((1, ch), lambda i: (0, 0)),
            pl.BlockSpec((1, ch), lambda i: (0, 0)),
            pl.BlockSpec((1, ch), lambda i: (0, 0)),
            pl.BlockSpec((1, c1), lambda i: (0, 0)),
            pl.BlockSpec((1, c1), lambda i: (0, 0)),
            pl.BlockSpec((c1, emb), lambda i: (0, 0)),
            pl.BlockSpec((1, emb), lambda i: (0, 0)),
            pl.BlockSpec((k, emb), lambda i: (0, 0)),
            pl.BlockSpec((1, k), lambda i: (0, 0)),
            pl.BlockSpec((emb, c1), lambda i: (0, 0)),
            pl.BlockSpec((1, c1), lambda i: (0, 0)),
        ],
        out_specs=pl.BlockSpec((_R, c1), lambda i: (i, 0)),
        out_shape=jax.ShapeDtypeStruct((_N, c1), _F32),
    )(ya, yb, sa, qa, sb, qb, g, b, wpre, bpre, cb, c2, wpost, bpost)


def _head_body(ya_ref, yb_ref, sa_ref, qa_ref, sb_ref, qb_ref, g_ref, b_ref,
               w1_ref, b1_ref, w2_ref, b2_ref, out_ref):
    y = jnp.concatenate([ya_ref[...], yb_ref[...]], axis=1)
    ssum = jnp.concatenate([sa_ref[...], sb_ref[...]], axis=1)
    ssq = jnp.concatenate([qa_ref[...], qb_ref[...]], axis=1)
    h = _gn_relu(y, ssum, ssq, g_ref[...], b_ref[...])
    mid = jnp.maximum(_bdot(h, w1_ref[...]) + b1_ref[...], 0.0)
    out_ref[...] = _bdot(mid, w2_ref[...]) + b2_ref[...]


def _head(ya, yb, sa, qa, sb, qb, g, b, w1, b1, w2, b2):
    c0 = w1.shape[0]
    mid = w1.shape[1]
    oc = w2.shape[1]
    ch = c0 // 2
    return pl.pallas_call(
        _head_body,
        grid=(_GRID,),
        in_specs=[
            pl.BlockSpec((_R, ch), lambda i: (i, 0)),
            pl.BlockSpec((_R, ch), lambda i: (i, 0)),
            pl.BlockSpec((1, ch), lambda i: (0, 0)),
            pl.BlockSpec((1, ch), lambda i: (0, 0)),
            pl.BlockSpec((1, ch), lambda i: (0, 0)),
            pl.BlockSpec((1, ch), lambda i: (0, 0)),
            pl.BlockSpec((1, c0), lambda i: (0, 0)),
            pl.BlockSpec((1, c0), lambda i: (0, 0)),
            pl.BlockSpec((c0, mid), lambda i: (0, 0)),
            pl.BlockSpec((1, mid), lambda i: (0, 0)),
            pl.BlockSpec((mid, oc), lambda i: (0, 0)),
            pl.BlockSpec((1, oc), lambda i: (0, 0)),
        ],
        out_specs=pl.BlockSpec((_R, oc), lambda i: (i, 0)),
        out_shape=jax.ShapeDtypeStruct((_N, oc), _F32),
    )(ya, yb, sa, qa, sb, qb, g, b, w1, b1, w2, b2)


@functools.cache
def _make_sc_conv(ch):
    mesh = plsc.VectorSubcoreMesh(core_axis_name="c", subcore_axis_name="s")
    grp = 1024 if ch <= 16 else 512
    nmacro = _SPS * _STREAM // grp
    nstream = grp // _STREAM

    @functools.partial(
        pl.kernel,
        out_type=[jax.ShapeDtypeStruct((_N, ch), _F32)] * 2,
        mesh=mesh,
        scratch_types=(
            [pltpu.VMEM((_STREAM,), jnp.int32)] * (2 * nstream)
            + [
                pltpu.VMEM((grp, ch), _F32),
                pltpu.VMEM_SHARED((_EPS, ch), _F32),
                pltpu.SemaphoreType.DMA,
            ]
        ),
        compiler_params=pltpu.CompilerParams(use_tc_tiling_on_sc=False),
    )
    def conv(xw0, xw1, gidx, dst, zrows, out0, out1, *scr):
        idxbs = scr[:nstream]
        dstbs = scr[nstream:2 * nstream]
        rows, acc, sem = scr[2 * nstream:]
        c = lax.axis_index("c")
        s = lax.axis_index("s")
        @pl.when(s < 15)
        def _():
            pltpu.sync_copy(zrows, acc.at[pl.ds(s * _RS_A, _RS_A)])

        @pl.when(s == 15)
        def _():
            pltpu.sync_copy(zrows.at[pl.ds(0, _RS_LAST)],
                            acc.at[pl.ds(15 * _RS_A, _RS_LAST)])

        plsc.subcore_barrier()

        def run(xw):
            def macro(i, carry):
                base = s * (_SPS * _STREAM) + i * grp
                for j in range(nstream):
                    pltpu.sync_copy(
                        gidx.at[pl.ds(base + j * _STREAM, _STREAM)], idxbs[j])
                    pltpu.sync_copy(
                        dst.at[pl.ds(base + j * _STREAM, _STREAM)], dstbs[j])
                cps = [
                    pltpu.async_copy(
                        xw.at[idxbs[j]],
                        rows.at[pl.ds(j * _STREAM, _STREAM)],
                        sem,
                    )
                    for j in range(nstream)
                ]
                for cp in cps:
                    cp.wait()
                for j in range(nstream):
                    pltpu.sync_copy(
                        rows.at[pl.ds(j * _STREAM, _STREAM)],
                        acc.at[dstbs[j]],
                        add=True,
                    )
                return carry
            lax.fori_loop(0, nmacro, macro, 0)

        @pl.when(c == 0)
        def _():
            run(xw0)

        @pl.when(c == 1)
        def _():
            run(xw1)

        plsc.subcore_barrier()

        def copy_out(out):
            @pl.when(s < 15)
            def _():
                pltpu.sync_copy(acc.at[pl.ds(s * _RS_A, _RS_A)],
                                out.at[pl.ds(s * _RS_A, _RS_A)])

            @pl.when(s == 15)
            def _():
                pltpu.sync_copy(acc.at[pl.ds(15 * _RS_A, _RS_LAST)],
                                out.at[pl.ds(15 * _RS_A, _RS_LAST)])

        @pl.when(c == 0)
        def _():
            copy_out(out0)

        @pl.when(c == 1)
        def _():
            copy_out(out1)

    return conv


def _sc_conv(ch, xw0, xw1, gidx, dstp, zrows):
    return _make_sc_conv(ch)(
        xw0.reshape(_N * _NET, ch), xw1.reshape(_N * _NET, ch),
        gidx, dstp, zrows)


def _halves(w):
    net, cin, cout = w.shape
    ch = cout // 2
    wt = jnp.transpose(w, (1, 0, 2))
    return (wt[:, :, :ch].reshape(cin, net * ch),
            wt[:, :, ch:].reshape(cin, net * ch))


def kernel(x, edge_index, edge_type, W_conv1, g1, b1, W_enc, g2, b2,
           W_down, g3, b3, W_pre, b_pre, codebook, W_post, b_post,
           W_dec1, g4, b4, W_dec2, g5, b5, W_reg1, b_reg1, W_reg2, b_reg2):
    src = edge_index[0].astype(jnp.int32)
    dst = edge_index[1].astype(jnp.int32)
    et = edge_type.astype(jnp.int32)

    pad = _EPAD - _E
    srcp = jnp.concatenate([src, jnp.arange(pad, dtype=jnp.int32) % 101])
    etp = jnp.concatenate([et, jnp.zeros((pad,), jnp.int32)])
    dstp = jnp.concatenate(
        [dst, _N + (jnp.arange(pad, dtype=jnp.int32) % _STREAM)])
    gidx = _compute_gidx(srcp, etp)
    z16 = jnp.zeros((_RS_A, 16), _F32)
    z32 = jnp.zeros((_RS_A, 32), _F32)

    r2 = lambda a: a.reshape(1, -1)

    w0, w1 = _halves(W_conv1)
    xw0, xw1 = _conv_in_plain(x, w0, w1)
    y1a, y1b = _sc_conv(16, xw0, xw1, gidx, dstp, z16)
    s1a, q1a = _stats(y1a, 1)
    s1b, q1b = _stats(y1b, 1)

    w0, w1 = _halves(W_enc)
    xw0, xw1, h1 = _conv_in_gn(y1a, y1b, s1a, q1a, s1b, q1b,
                               r2(g1), r2(b1), w0, w1)
    y2a, y2b = _sc_conv(16, xw0, xw1, gidx, dstp, z16)
    s2a, q2a = _stats(y2a, 1)
    s2b, q2b = _stats(y2b, 1)

    w0, w1 = _halves(W_down)
    xw0, xw1 = _conv_in_res(h1, y2a, y2b, s2a, q2a, s2b, q2b,
                            r2(g2), r2(b2), w0, w1)
    y3a, y3b = _sc_conv(32, xw0, xw1, gidx, dstp, z32)
    s3a, q3a = _stats(y3a, 2)
    s3b, q3b = _stats(y3b, 2)

    c2 = jnp.sum(codebook * codebook, axis=1).reshape(1, -1)
    h4 = _vq(y3a, y3b, s3a, q3a, s3b, q3b, r2(g3), r2(b3),
             W_pre, r2(b_pre), codebook, c2, W_post, r2(b_post))

    w0, w1 = _halves(W_dec1)
    xw0, xw1 = _conv_in_plain(h4, w0, w1)
    y4a, y4b = _sc_conv(32, xw0, xw1, gidx, dstp, z32)
    s4a, q4a = _stats(y4a, 2)
    s4b, q4b = _stats(y4b, 2)

    w0, w1 = _halves(W_dec2)
    xw0, xw1 = _conv_in_res(h4, y4a, y4b, s4a, q4a, s4b, q4b,
                            r2(g4), r2(b4), w0, w1)
    y5a, y5b = _sc_conv(16, xw0, xw1, gidx, dstp, z16)
    s5a, q5a = _stats(y5a, 1)
    s5b, q5b = _stats(y5b, 1)

    return _head(y5a, y5b, s5a, q5a, s5b, q5b, r2(g5), r2(b5),
                 W_reg1, r2(b_reg1), W_reg2, r2(b_reg2))
